# baseline (speedup 1.0000x reference)
import functools

import jax
import jax.numpy as jnp
from jax import lax
from jax.experimental import pallas as pl
from jax.experimental.pallas import tpu as pltpu
from jax.experimental.pallas import tpu_sc as plsc

BATCH = 16384
EMBED_DIM = 64
DENSE_IN = 384
OUT_DIM = 4 * EMBED_DIM

NC = 2
NS = 16
NW = NC * NS
ROWS_W = BATCH // NW
CHUNK = 64
N_CHUNKS = ROWS_W // CHUNK
NBUF = 2


def _matmul_body(x_ref, w_ref, b_ref, o_ref):
    o_ref[...] = (
        jnp.dot(x_ref[...], w_ref[...], preferred_element_type=jnp.float32)
        + b_ref[...]
    )


def _dense_matmul(x, W, b):
    block_rows = 1024
    return pl.pallas_call(
        _matmul_body,
        grid=(BATCH // block_rows,),
        in_specs=[
            pl.BlockSpec((block_rows, DENSE_IN), lambda i: (i, 0)),
            pl.BlockSpec((DENSE_IN, EMBED_DIM), lambda i: (0, 0)),
            pl.BlockSpec((1, EMBED_DIM), lambda i: (0, 0)),
        ],
        out_specs=pl.BlockSpec((block_rows, EMBED_DIM), lambda i: (i, 0)),
        out_shape=jax.ShapeDtypeStruct((BATCH, EMBED_DIM), jnp.float32),
    )(x, W, b.reshape(1, EMBED_DIM))


def _sc_body(aidx, aoff, didx, doff, pidx, poff, atab2, dtab2, ptab2, dense2,
             out, idxs_v, offs_v, abuf, dbuf, pbuf, xbuf, asm,
             gsem0, gsem1, wsem0, wsem1):
    wid = lax.axis_index("s") * NC + lax.axis_index("c")
    base = wid * ROWS_W
    r0 = wid * N_CHUNKS
    gsems = (gsem0, gsem1)
    wsems = (wsem0, wsem1)

    pltpu.sync_copy(aidx.at[pl.ds(r0, N_CHUNKS)], idxs_v.at[0])
    pltpu.sync_copy(didx.at[pl.ds(r0, N_CHUNKS)], idxs_v.at[1])
    pltpu.sync_copy(pidx.at[pl.ds(r0, N_CHUNKS)], idxs_v.at[2])
    pltpu.sync_copy(aoff.at[pl.ds(r0, N_CHUNKS)], offs_v.at[0])
    pltpu.sync_copy(doff.at[pl.ds(r0, N_CHUNKS)], offs_v.at[1])
    pltpu.sync_copy(poff.at[pl.ds(r0, N_CHUNKS)], offs_v.at[2])

    lane = lax.iota(jnp.int32, 16)

    def fire_gathers(j):
        s = j % NBUF
        sem = gsems[s]
        return [
            pltpu.async_copy(atab2.at[idxs_v.at[0].at[j]], abuf.at[s], sem),
            pltpu.async_copy(dtab2.at[idxs_v.at[1].at[j]], dbuf.at[s], sem),
            pltpu.async_copy(ptab2.at[idxs_v.at[2].at[j]], pbuf.at[s], sem),
            pltpu.async_copy(
                dense2.at[pl.ds(
                    pl.multiple_of(
                        wid * (ROWS_W // 2) + j * (CHUNK // 2), 8
                    ),
                    CHUNK // 2,
                )],
                xbuf.at[s],
                sem,
            ),
        ]

    def assemble(j):
        s = j % NBUF
        panels = (
            (abuf.at[s], offs_v.at[0].at[j]),
            (dbuf.at[s], offs_v.at[1].at[j]),
            (pbuf.at[s], offs_v.at[2].at[j]),
        )
        asm_s = asm.at[s]
        xbuf_s = xbuf.at[s]

        def pair_body(p, carry):
            for dr in range(2):
                i = 2 * p + dr
                i_vec = jnp.zeros((16,), jnp.int32) + i
                for t, (buf, offr) in enumerate(panels):
                    off_vec = plsc.load_gather(offr, [i_vec])
                    for c in range(EMBED_DIM // 16):
                        col = off_vec + (lane + c * 16)
                        asm_s[i, pl.ds(t * EMBED_DIM + c * 16, 16)] = (
                            plsc.load_gather(buf, [i_vec, col])
                        )
                for c in range(EMBED_DIM // 16):
                    asm_s[i, pl.ds(3 * EMBED_DIM + c * 16, 16)] = (
                        xbuf_s[p, pl.ds(dr * EMBED_DIM + c * 16, 16)]
                    )
            return carry

        lax.fori_loop(0, CHUNK // 2, pair_body, 0)

    g_handles = {0: fire_gathers(0)}
    w_handles = {}
    for j in range(N_CHUNKS):
        if j + 1 < N_CHUNKS:
            g_handles[j + 1] = fire_gathers(j + 1)
        for h in g_handles.pop(j):
            h.wait()
        if j - NBUF in w_handles:
            w_handles.pop(j - NBUF).wait()
        assemble(j)
        w_handles[j] = pltpu.async_copy(
            asm.at[j % NBUF],
            out.at[pl.ds(base + j * CHUNK, CHUNK)],
            wsems[j % NBUF],
        )
    for h in w_handles.values():
        h.wait()


_sc_assemble = functools.partial(
    pl.kernel,
    out_type=jax.ShapeDtypeStruct((BATCH, OUT_DIM), jnp.float32),
    mesh=plsc.VectorSubcoreMesh(
        core_axis_name="c", subcore_axis_name="s", num_cores=NC, num_subcores=NS
    ),
    scratch_types=[
        pltpu.VMEM((3, N_CHUNKS, CHUNK), jnp.int32),
        pltpu.VMEM((3, N_CHUNKS, CHUNK), jnp.int32),
        pltpu.VMEM((NBUF, CHUNK, 128), jnp.float32),
        pltpu.VMEM((NBUF, CHUNK, 128), jnp.float32),
        pltpu.VMEM((NBUF, CHUNK, 128), jnp.float32),
        pltpu.VMEM((NBUF, CHUNK // 2, 128), jnp.float32),
        pltpu.VMEM((NBUF, CHUNK, OUT_DIM), jnp.float32),
        pltpu.SemaphoreType.DMA,
        pltpu.SemaphoreType.DMA,
        pltpu.SemaphoreType.DMA,
        pltpu.SemaphoreType.DMA,
    ],
    compiler_params=pltpu.CompilerParams(needs_layout_passes=False),
)(_sc_body)


def _split_idx(i):
    i = i.astype(jnp.int32)
    half = (i >> 1).reshape(BATCH // CHUNK, CHUNK)
    off = ((i & 1) << 6).reshape(BATCH // CHUNK, CHUNK)
    return half, off


def _pair(t):
    return jnp.concatenate([t[0::2], t[1::2]], axis=1)


def kernel(book_ids, book_authors, book_dates, book_publishers,
           book_title_embeddings, id_table, author_table, date_table,
           publisher_table, W, b):
    dense = _dense_matmul(book_title_embeddings, W, b)
    aidx, aoff = _split_idx(book_authors)
    didx, doff = _split_idx(book_dates)
    pidx, poff = _split_idx(book_publishers)
    return _sc_assemble(
        aidx, aoff, didx, doff, pidx, poff,
        _pair(author_table),
        _pair(date_table),
        _pair(publisher_table),
        _pair(dense),
    )


# speedup vs baseline: 9.7999x; 9.7999x over previous
import functools

import jax
import jax.numpy as jnp
from jax import lax
from jax.experimental import pallas as pl
from jax.experimental.pallas import tpu as pltpu
from jax.experimental.pallas import tpu_sc as plsc

BATCH = 16384
EMBED_DIM = 64
DENSE_IN = 384
OUT_DIM = 4 * EMBED_DIM

NC = 2
NS = 16
NW = NC * NS
ROWS_W = BATCH // NW
CHUNK = 64
N_CHUNKS = ROWS_W // CHUNK
NBUF = 2


def _matmul_body(x_ref, w_ref, b_ref, o_ref):
    o_ref[...] = (
        jnp.dot(x_ref[...], w_ref[...], preferred_element_type=jnp.float32)
        + b_ref[...]
    )


def _dense_matmul(x, W, b):
    block_rows = 1024
    return pl.pallas_call(
        _matmul_body,
        grid=(BATCH // block_rows,),
        in_specs=[
            pl.BlockSpec((block_rows, DENSE_IN), lambda i: (i, 0)),
            pl.BlockSpec((DENSE_IN, EMBED_DIM), lambda i: (0, 0)),
            pl.BlockSpec((1, EMBED_DIM), lambda i: (0, 0)),
        ],
        out_specs=pl.BlockSpec((block_rows, EMBED_DIM), lambda i: (i, 0)),
        out_shape=jax.ShapeDtypeStruct((BATCH, EMBED_DIM), jnp.float32),
    )(x, W, b.reshape(1, EMBED_DIM))


def _sc_body(aidx, aoff, didx, doff, pidx, poff, atab2, dtab2, ptab2, dense,
             out, idxs_v, offs_v, abuf, dbuf, pbuf, xbuf, asm,
             gsem0, gsem1, wsem0, wsem1):
    wid = lax.axis_index("s") * NC + lax.axis_index("c")
    base = wid * ROWS_W
    r0 = wid * N_CHUNKS
    gsems = (gsem0, gsem1)
    wsems = (wsem0, wsem1)

    pltpu.sync_copy(aidx.at[pl.ds(r0, N_CHUNKS)], idxs_v.at[0])
    pltpu.sync_copy(didx.at[pl.ds(r0, N_CHUNKS)], idxs_v.at[1])
    pltpu.sync_copy(pidx.at[pl.ds(r0, N_CHUNKS)], idxs_v.at[2])
    pltpu.sync_copy(aoff.at[pl.ds(r0, N_CHUNKS)], offs_v.at[0])
    pltpu.sync_copy(doff.at[pl.ds(r0, N_CHUNKS)], offs_v.at[1])
    pltpu.sync_copy(poff.at[pl.ds(r0, N_CHUNKS)], offs_v.at[2])

    lane = lax.iota(jnp.int32, 16)

    def fire_gathers(j):
        s = j % NBUF
        sem = gsems[s]
        return [
            pltpu.async_copy(atab2.at[idxs_v.at[0].at[j]], abuf.at[s], sem),
            pltpu.async_copy(dtab2.at[idxs_v.at[1].at[j]], dbuf.at[s], sem),
            pltpu.async_copy(ptab2.at[idxs_v.at[2].at[j]], pbuf.at[s], sem),
            pltpu.async_copy(
                dense.at[pl.ds(
                    pl.multiple_of(base + j * CHUNK, 8), CHUNK
                )],
                xbuf.at[s],
                sem,
            ),
        ]

    def assemble(j):
        s = j % NBUF
        panels = (
            (abuf.at[s], offs_v.at[0].at[j]),
            (dbuf.at[s], offs_v.at[1].at[j]),
            (pbuf.at[s], offs_v.at[2].at[j]),
        )
        asm_s = asm.at[s]
        xbuf_s = xbuf.at[s]

        def pair_body(p, carry):
            for dr in range(2):
                i = 2 * p + dr
                i_vec = jnp.zeros((16,), jnp.int32) + i
                for t, (buf, offr) in enumerate(panels):
                    off_vec = plsc.load_gather(offr, [i_vec])
                    for c in range(EMBED_DIM // 16):
                        col = off_vec + (lane + c * 16)
                        asm_s[i, pl.ds(t * EMBED_DIM + c * 16, 16)] = (
                            plsc.load_gather(buf, [i_vec, col])
                        )
                for c in range(EMBED_DIM // 16):
                    asm_s[i, pl.ds(3 * EMBED_DIM + c * 16, 16)] = (
                        xbuf_s[i, pl.ds(c * 16, 16)]
                    )
            return carry

        lax.fori_loop(0, CHUNK // 2, pair_body, 0)

    g_handles = {0: fire_gathers(0)}
    w_handles = {}
    for j in range(N_CHUNKS):
        if j + 1 < N_CHUNKS:
            g_handles[j + 1] = fire_gathers(j + 1)
        for h in g_handles.pop(j):
            h.wait()
        if j - NBUF in w_handles:
            w_handles.pop(j - NBUF).wait()
        assemble(j)
        w_handles[j] = pltpu.async_copy(
            asm.at[j % NBUF],
            out.at[pl.ds(base + j * CHUNK, CHUNK)],
            wsems[j % NBUF],
        )
    for h in w_handles.values():
        h.wait()


_sc_assemble = functools.partial(
    pl.kernel,
    out_type=jax.ShapeDtypeStruct((BATCH, OUT_DIM), jnp.float32),
    mesh=plsc.VectorSubcoreMesh(
        core_axis_name="c", subcore_axis_name="s", num_cores=NC, num_subcores=NS
    ),
    scratch_types=[
        pltpu.VMEM((3, N_CHUNKS, CHUNK), jnp.int32),
        pltpu.VMEM((3, N_CHUNKS, CHUNK), jnp.int32),
        pltpu.VMEM((NBUF, CHUNK, 128), jnp.float32),
        pltpu.VMEM((NBUF, CHUNK, 128), jnp.float32),
        pltpu.VMEM((NBUF, CHUNK, 128), jnp.float32),
        pltpu.VMEM((NBUF, CHUNK, EMBED_DIM), jnp.float32),
        pltpu.VMEM((NBUF, CHUNK, OUT_DIM), jnp.float32),
        pltpu.SemaphoreType.DMA,
        pltpu.SemaphoreType.DMA,
        pltpu.SemaphoreType.DMA,
        pltpu.SemaphoreType.DMA,
    ],
    compiler_params=pltpu.CompilerParams(needs_layout_passes=False),
)(_sc_body)


def _split_idx(i):
    i = i.astype(jnp.int32)
    half = (i >> 1).reshape(BATCH // CHUNK, CHUNK)
    off = ((i & 1) << 6).reshape(BATCH // CHUNK, CHUNK)
    return half, off


def kernel(book_ids, book_authors, book_dates, book_publishers,
           book_title_embeddings, id_table, author_table, date_table,
           publisher_table, W, b):
    dense = _dense_matmul(book_title_embeddings, W, b)
    aidx, aoff = _split_idx(book_authors)
    didx, doff = _split_idx(book_dates)
    pidx, poff = _split_idx(book_publishers)
    return _sc_assemble(
        aidx, aoff, didx, doff, pidx, poff,
        author_table.reshape(-1, 2 * EMBED_DIM),
        date_table.reshape(-1, 2 * EMBED_DIM),
        publisher_table.reshape(-1, 2 * EMBED_DIM),
        dense,
    )
